# baseline (speedup 1.0000x reference)
import functools
import math

import jax
import jax.numpy as jnp
from jax.experimental import pallas as pl
from jax.experimental.pallas import tpu as pltpu

_NZ = 100
_NL = 100000
_NU = 1024
_BK = 2048
_NBLK = (_NL + _BK - 1) // _BK
_KA = 128
_LOG_NORM = 0.5 * _NZ * math.log(2.0 * math.pi)


def _dd_body(B_ref, A_ref, o_ref, tmin_ref):
    i = pl.program_id(0)
    Ab = A_ref[...]
    B = B_ref[...]
    t = jax.lax.dot_general(
        Ab, B.astype(jnp.bfloat16), (((1,), (0,)), ((), ())),
        preferred_element_type=jnp.float32,
    )

    @pl.when(i < _NBLK - 1)
    def _():
        bmin = jnp.min(t, axis=0, keepdims=True)
        tmin_ref[...] = jnp.where(i == 0, bmin,
                                  jnp.minimum(tmin_ref[...], bmin))

    @pl.when(i == _NBLK - 1)
    def _():
        gidx = i * _BK + jax.lax.broadcasted_iota(jnp.int32, (_BK, 1), 0)
        bmin = jnp.min(jnp.where(gidx < _NL, t, jnp.inf),
                       axis=0, keepdims=True)
        tmin = jnp.minimum(tmin_ref[...], bmin)
        U2 = 0.25 * jnp.sum(B[:_NZ, :] * B[:_NZ, :], axis=0,
                            keepdims=True)
        d2 = jnp.maximum(tmin + U2, 0.0)
        div = jnp.log(jnp.sqrt(d2) + 1e-18)
        dens = -0.5 * U2 - _LOG_NORM
        dd = jnp.exp(dens + div)
        dd = dd - jnp.min(dd)
        o_ref[...] = dd / (jnp.max(dd) + 1e-18)


@functools.partial(jax.jit, static_argnames=("interpret",))
def _dd_call(B, A, interpret=False):
    return pl.pallas_call(
        _dd_body,
        grid=(_NBLK,),
        in_specs=[
            pl.BlockSpec((_KA, _NU), lambda i: (0, 0)),
            pl.BlockSpec((_BK, _KA), lambda i: (i, 0)),
        ],
        out_specs=pl.BlockSpec((1, _NU), lambda i: (0, 0)),
        out_shape=jax.ShapeDtypeStruct((1, _NU), jnp.float32),
        scratch_shapes=[pltpu.VMEM((1, _NU), jnp.float32)],
        compiler_params=pltpu.CompilerParams(
            dimension_semantics=("arbitrary",),
        ),
        interpret=interpret,
    )(B, A)


def kernel(pred, U_z, L_z):
    del pred
    L2 = jnp.sum(L_z * L_z, axis=1, keepdims=True)
    A = jnp.concatenate(
        [L_z, L2, jnp.zeros((_NL, _KA - _NZ - 1), jnp.float32)],
        axis=1).astype(jnp.bfloat16)
    B = jnp.concatenate(
        [-2.0 * U_z.T, jnp.ones((1, _NU), jnp.float32),
         jnp.zeros((_KA - _NZ - 1, _NU), jnp.float32)], axis=0)
    out = _dd_call(B, A)
    return out.reshape(-1)


# speedup vs baseline: 1.4853x; 1.4853x over previous
import functools
import math

import jax
import jax.numpy as jnp
from jax.experimental import pallas as pl
from jax.experimental.pallas import tpu as pltpu

_NZ = 100
_NL = 100000
_NU = 1024
_BK = 2000
_NBLK = _NL // _BK
_LOG_NORM = 0.5 * _NZ * math.log(2.0 * math.pi)


def _min_body(B_ref, L_ref, o_ref, P_ref, L2_ref):
    i = pl.program_id(0)
    slot = jax.lax.rem(i, 2)

    @pl.when(i < _NBLK)
    def _():
        Lb = L_ref[...]
        P_ref[slot] = jax.lax.dot_general(
            Lb, B_ref[...], (((1,), (0,)), ((), ())),
            preferred_element_type=jnp.float32,
        )
        L2_ref[slot] = jnp.sum(Lb * Lb, axis=1, keepdims=True)

    @pl.when(i > 0)
    def _():
        prev = 1 - slot
        t = L2_ref[prev] + P_ref[prev]
        bmin = jnp.min(t, axis=0, keepdims=True)
        o_ref[...] = jnp.where(i == 1, bmin,
                               jnp.minimum(o_ref[...], bmin))


def _fin_body(B_ref, tmin_ref, o_ref):
    B = B_ref[...]
    U2 = 0.25 * jnp.sum(B * B, axis=0, keepdims=True)
    d2 = jnp.maximum(tmin_ref[...] + U2, 0.0)
    div = jnp.log(jnp.sqrt(d2) + 1e-18)
    dens = -0.5 * U2 - _LOG_NORM
    dd = jnp.exp(dens + div)
    dd = dd - jnp.min(dd)
    o_ref[...] = dd / (jnp.max(dd) + 1e-18)


@functools.partial(jax.jit, static_argnames=("interpret",))
def _dd_call(B, L_z, interpret=False):
    tmin = pl.pallas_call(
        _min_body,
        grid=(_NBLK + 1,),
        in_specs=[
            pl.BlockSpec((_NZ, _NU), lambda i: (0, 0)),
            pl.BlockSpec((_BK, _NZ), lambda i: (jnp.minimum(i, _NBLK - 1), 0)),
        ],
        out_specs=pl.BlockSpec((1, _NU), lambda i: (0, 0)),
        out_shape=jax.ShapeDtypeStruct((1, _NU), jnp.float32),
        scratch_shapes=[
            pltpu.VMEM((2, _BK, _NU), jnp.float32),
            pltpu.VMEM((2, _BK, 1), jnp.float32),
        ],
        compiler_params=pltpu.CompilerParams(
            dimension_semantics=("arbitrary",),
        ),
        interpret=interpret,
    )(B, L_z)
    return pl.pallas_call(
        _fin_body,
        out_shape=jax.ShapeDtypeStruct((1, _NU), jnp.float32),
        interpret=interpret,
    )(B, tmin)


def kernel(pred, U_z, L_z):
    del pred
    out = _dd_call(-2.0 * U_z.T, L_z)
    return out.reshape(-1)


# speedup vs baseline: 1.9654x; 1.3233x over previous
import functools
import math

import jax
import jax.numpy as jnp
from jax.experimental import pallas as pl
from jax.experimental.pallas import tpu as pltpu

_NZ = 100
_NL = 100000
_NU = 1024
_NS = 4
_BK = 1000
_NBLK = _NL // (_NS * _BK)
_LOG_NORM = 0.5 * _NZ * math.log(2.0 * math.pi)


def _min_body(B_ref, L0_ref, L1_ref, L2_ref, L3_ref, o_ref):
    i = pl.program_id(0)
    B = B_ref[...]
    bmin = None
    for Lr in (L0_ref, L1_ref, L2_ref, L3_ref):
        Lb = Lr[...]
        P = jax.lax.dot_general(
            Lb, B, (((1,), (0,)), ((), ())),
            preferred_element_type=jnp.float32,
        )
        l2 = jnp.sum(Lb * Lb, axis=1, keepdims=True)
        m = jnp.min(l2 + P, axis=0, keepdims=True)
        bmin = m if bmin is None else jnp.minimum(bmin, m)
    o_ref[...] = jnp.where(i == 0, bmin, jnp.minimum(o_ref[...], bmin))


def _fin_body(B_ref, tmin_ref, o_ref):
    B = B_ref[...]
    U2 = 0.25 * jnp.sum(B * B, axis=0, keepdims=True)
    d2 = jnp.maximum(tmin_ref[...] + U2, 0.0)
    div = jnp.log(jnp.sqrt(d2) + 1e-18)
    dens = -0.5 * U2 - _LOG_NORM
    dd = jnp.exp(dens + div)
    dd = dd - jnp.min(dd)
    o_ref[...] = dd / (jnp.max(dd) + 1e-18)


def _stream_spec(k):
    return pl.BlockSpec((_BK, _NZ), lambda i, k=k: (k * _NBLK + i, 0))


@functools.partial(jax.jit, static_argnames=("interpret",))
def _dd_call(B, L_z, interpret=False):
    tmin = pl.pallas_call(
        _min_body,
        grid=(_NBLK,),
        in_specs=[pl.BlockSpec((_NZ, _NU), lambda i: (0, 0))]
        + [_stream_spec(k) for k in range(_NS)],
        out_specs=pl.BlockSpec((1, _NU), lambda i: (0, 0)),
        out_shape=jax.ShapeDtypeStruct((1, _NU), jnp.float32),
        compiler_params=pltpu.CompilerParams(
            dimension_semantics=("arbitrary",),
        ),
        interpret=interpret,
    )(B, L_z, L_z, L_z, L_z)
    return pl.pallas_call(
        _fin_body,
        out_shape=jax.ShapeDtypeStruct((1, _NU), jnp.float32),
        interpret=interpret,
    )(B, tmin)


def kernel(pred, U_z, L_z):
    del pred
    out = _dd_call(-2.0 * U_z.T, L_z)
    return out.reshape(-1)


# speedup vs baseline: 1.9783x; 1.0066x over previous
import functools
import math

import jax
import jax.numpy as jnp
from jax.experimental import pallas as pl
from jax.experimental.pallas import tpu as pltpu

_NZ = 100
_NL = 100000
_NU = 1024
_NS = 5
_BK = 1000
_NBLK = _NL // (_NS * _BK)
_LOG_NORM = 0.5 * _NZ * math.log(2.0 * math.pi)


def _min_body(B_ref, L0_ref, L1_ref, L2_ref, L3_ref, L4_ref, o_ref):
    i = pl.program_id(0)
    B = B_ref[...]
    bmin = None
    for Lr in (L0_ref, L1_ref, L2_ref, L3_ref, L4_ref):
        Lb = Lr[...]
        P = jax.lax.dot_general(
            Lb, B, (((1,), (0,)), ((), ())),
            preferred_element_type=jnp.float32,
        )
        l2 = jnp.sum(Lb * Lb, axis=1, keepdims=True)
        m = jnp.min(l2 + P, axis=0, keepdims=True)
        bmin = m if bmin is None else jnp.minimum(bmin, m)
    o_ref[...] = jnp.where(i == 0, bmin, jnp.minimum(o_ref[...], bmin))


def _fin_body(B_ref, tmin_ref, o_ref):
    B = B_ref[...]
    U2 = 0.25 * jnp.sum(B * B, axis=0, keepdims=True)
    d2 = jnp.maximum(tmin_ref[...] + U2, 0.0)
    div = jnp.log(jnp.sqrt(d2) + 1e-18)
    dens = -0.5 * U2 - _LOG_NORM
    dd = jnp.exp(dens + div)
    dd = dd - jnp.min(dd)
    o_ref[...] = dd / (jnp.max(dd) + 1e-18)


def _stream_spec(k):
    return pl.BlockSpec((_BK, _NZ), lambda i, k=k: (k * _NBLK + i, 0))


@functools.partial(jax.jit, static_argnames=("interpret",))
def _dd_call(B, L_z, interpret=False):
    tmin = pl.pallas_call(
        _min_body,
        grid=(_NBLK,),
        in_specs=[pl.BlockSpec((_NZ, _NU), lambda i: (0, 0))]
        + [_stream_spec(k) for k in range(_NS)],
        out_specs=pl.BlockSpec((1, _NU), lambda i: (0, 0)),
        out_shape=jax.ShapeDtypeStruct((1, _NU), jnp.float32),
        compiler_params=pltpu.CompilerParams(
            dimension_semantics=("arbitrary",),
        ),
        interpret=interpret,
    )(B, L_z, L_z, L_z, L_z, L_z)
    return pl.pallas_call(
        _fin_body,
        out_shape=jax.ShapeDtypeStruct((1, _NU), jnp.float32),
        interpret=interpret,
    )(B, tmin)


def kernel(pred, U_z, L_z):
    del pred
    out = _dd_call(-2.0 * U_z.T, L_z)
    return out.reshape(-1)


# speedup vs baseline: 1.9948x; 1.0083x over previous
import functools
import math

import jax
import jax.numpy as jnp
from jax.experimental import pallas as pl
from jax.experimental.pallas import tpu as pltpu

_NZ = 100
_NL = 100000
_NU = 1024
_NS = 10
_BK = 1000
_NBLK = _NL // (_NS * _BK)
_LOG_NORM = 0.5 * _NZ * math.log(2.0 * math.pi)


def _min_body(B_ref, *refs):
    (L_refs, o_ref) = (refs[:-1], refs[-1])
    i = pl.program_id(0)
    B = B_ref[...]
    bmin = None
    for Lr in L_refs:
        Lb = Lr[...]
        P = jax.lax.dot_general(
            Lb, B, (((1,), (0,)), ((), ())),
            preferred_element_type=jnp.float32,
        )
        l2 = jnp.sum(Lb * Lb, axis=1, keepdims=True)
        m = jnp.min(l2 + P, axis=0, keepdims=True)
        bmin = m if bmin is None else jnp.minimum(bmin, m)
    o_ref[...] = jnp.where(i == 0, bmin, jnp.minimum(o_ref[...], bmin))


def _fin_body(B_ref, tmin_ref, o_ref):
    B = B_ref[...]
    U2 = 0.25 * jnp.sum(B * B, axis=0, keepdims=True)
    d2 = jnp.maximum(tmin_ref[...] + U2, 0.0)
    div = jnp.log(jnp.sqrt(d2) + 1e-18)
    dens = -0.5 * U2 - _LOG_NORM
    dd = jnp.exp(dens + div)
    dd = dd - jnp.min(dd)
    o_ref[...] = dd / (jnp.max(dd) + 1e-18)


def _stream_spec(k):
    return pl.BlockSpec((_BK, _NZ), lambda i, k=k: (k * _NBLK + i, 0))


@functools.partial(jax.jit, static_argnames=("interpret",))
def _dd_call(B, L_z, interpret=False):
    tmin = pl.pallas_call(
        _min_body,
        grid=(_NBLK,),
        in_specs=[pl.BlockSpec((_NZ, _NU), lambda i: (0, 0))]
        + [_stream_spec(k) for k in range(_NS)],
        out_specs=pl.BlockSpec((1, _NU), lambda i: (0, 0)),
        out_shape=jax.ShapeDtypeStruct((1, _NU), jnp.float32),
        compiler_params=pltpu.CompilerParams(
            dimension_semantics=("arbitrary",),
        ),
        interpret=interpret,
    )(B, *([L_z] * _NS))
    return pl.pallas_call(
        _fin_body,
        out_shape=jax.ShapeDtypeStruct((1, _NU), jnp.float32),
        interpret=interpret,
    )(B, tmin)


def kernel(pred, U_z, L_z):
    del pred
    out = _dd_call(-2.0 * U_z.T, L_z)
    return out.reshape(-1)
